# baseline (speedup 1.0000x reference)
import functools

import jax
import jax.numpy as jnp
from jax import lax
from jax.experimental import pallas as pl
from jax.experimental.pallas import tpu as pltpu
from jax.experimental.pallas import tpu_sc as plsc

D_MODEL = 128
SEQ_LEN = 200
CHUNK = 128
NBUF = 5
TC_SEQS = 8
EPS = 1e-5


def _make_sc_gather(n_tok):
    info = plsc.get_sparse_core_info()
    nw = info.num_cores * info.num_subcores
    tok_per_w = n_tok // nw
    n_chunks = tok_per_w // CHUNK
    assert n_chunks % NBUF == 0
    mesh = plsc.VectorSubcoreMesh(core_axis_name="c", subcore_axis_name="s")

    @functools.partial(
        pl.kernel,
        out_type=jax.ShapeDtypeStruct((n_tok, D_MODEL), jnp.float32),
        mesh=mesh,
        scratch_types=[
            pltpu.VMEM((n_chunks, CHUNK), jnp.int32),
            pltpu.VMEM((NBUF, CHUNK, D_MODEL), jnp.float32),
            pltpu.SemaphoreType.DMA((NBUF,)),
            pltpu.SemaphoreType.DMA((NBUF,)),
        ],
    )
    def sc_gather(x_hbm, table_hbm, out_hbm, idx_v, bufs, gsem, osem):
        wid = lax.axis_index("s") * info.num_cores + lax.axis_index("c")
        base = wid * tok_per_w
        pltpu.sync_copy(x_hbm.at[wid], idx_v)

        def out_slice(j):
            return out_hbm.at[pl.ds(base + j * CHUNK, CHUNK)]

        def ring_round(g, carry):
            cps = []
            for b in range(NBUF):
                @pl.when(g > 0)
                def _wait_prev():
                    pltpu.make_async_copy(
                        bufs.at[b], out_slice(g - NBUF + b), osem.at[b]
                    ).wait()
                cps.append(pltpu.async_copy(
                    table_hbm.at[idx_v.at[g + b]], bufs.at[b], gsem.at[b]))
            for b in range(NBUF):
                cps[b].wait()
                pltpu.async_copy(bufs.at[b], out_slice(g + b), osem.at[b])
            return carry

        lax.fori_loop(0, n_chunks // NBUF, lambda i, c: ring_round(i * NBUF, c),
                      0, unroll=False)
        for b in range(NBUF):
            pltpu.make_async_copy(
                bufs.at[b], out_slice(n_chunks - NBUF + b), osem.at[b]).wait()

    return sc_gather


def _tc_ln_body(emb_ref, pe_ref, g_ref, b_ref, out_ref):
    e = emb_ref[...] + pe_ref[...]
    ones = jnp.full((D_MODEL, D_MODEL), 1.0 / D_MODEL, dtype=jnp.float32)
    mean = lax.dot(e, ones, precision=lax.Precision.HIGHEST)
    c = e - mean
    var = lax.dot(c * c, ones, precision=lax.Precision.HIGHEST)
    out_ref[...] = c * lax.rsqrt(var + EPS) * g_ref[...] + b_ref[...]


def _tc_ln(emb, pe_big, gamma, beta):
    n_tok = emb.shape[0]
    rows = TC_SEQS * SEQ_LEN
    grid = n_tok // rows
    return pl.pallas_call(
        _tc_ln_body,
        grid=(grid,),
        in_specs=[
            pl.BlockSpec((rows, D_MODEL), lambda i: (i, 0)),
            pl.BlockSpec((rows, D_MODEL), lambda i: (0, 0)),
            pl.BlockSpec((1, D_MODEL), lambda i: (0, 0)),
            pl.BlockSpec((1, D_MODEL), lambda i: (0, 0)),
        ],
        out_specs=pl.BlockSpec((rows, D_MODEL), lambda i: (i, 0)),
        out_shape=jax.ShapeDtypeStruct((n_tok, D_MODEL), jnp.float32),
    )(emb, pe_big, gamma, beta)


def kernel(x, table, pe, gamma, beta):
    n_seq, seq_len = x.shape
    assert seq_len == SEQ_LEN
    n_tok = n_seq * seq_len
    info = plsc.get_sparse_core_info()
    nw = info.num_cores * info.num_subcores
    xc = x.astype(jnp.int32).reshape(nw, n_tok // nw // CHUNK, CHUNK)
    gathered = _make_sc_gather(n_tok)(xc, table)
    pe_big = jnp.tile(pe[0, :SEQ_LEN, :], (TC_SEQS, 1))
    out = _tc_ln(gathered, pe_big, gamma.reshape(1, D_MODEL),
                 beta.reshape(1, D_MODEL))
    return out.reshape(n_seq, seq_len, D_MODEL)


# speedup vs baseline: 1.8889x; 1.8889x over previous
import functools

import jax
import jax.numpy as jnp
from jax import lax
from jax.experimental import pallas as pl
from jax.experimental.pallas import tpu as pltpu
from jax.experimental.pallas import tpu_sc as plsc

D_MODEL = 128
SEQ_LEN = 200
CHUNK = 128
NBUF = 5
TC_SEQS = 8
EPS = 1e-5


def _make_sc_gather(n_tok):
    info = plsc.get_sparse_core_info()
    nw = info.num_cores * info.num_subcores
    tok_per_w = n_tok // nw
    n_chunks = tok_per_w // CHUNK
    assert n_chunks % NBUF == 0
    mesh = plsc.VectorSubcoreMesh(core_axis_name="c", subcore_axis_name="s")

    @functools.partial(
        pl.kernel,
        out_type=jax.ShapeDtypeStruct((n_tok, D_MODEL), jnp.float32),
        mesh=mesh,
        scratch_types=[
            pltpu.VMEM((n_chunks, CHUNK), jnp.int32),
            pltpu.VMEM((NBUF, CHUNK, D_MODEL), jnp.float32),
            pltpu.SemaphoreType.DMA((NBUF,)),
            pltpu.SemaphoreType.DMA((NBUF,)),
        ],
    )
    def sc_gather(x_hbm, table_hbm, out_hbm, idx_v, bufs, gsem, osem):
        wid = lax.axis_index("s") * info.num_cores + lax.axis_index("c")
        base = wid * tok_per_w
        pltpu.sync_copy(x_hbm.at[wid], idx_v)

        def out_slice(j):
            return out_hbm.at[pl.ds(base + j * CHUNK, CHUNK)]

        def ring_round(g, carry):
            cps = []
            for b in range(NBUF):
                @pl.when(g > 0)
                def _wait_prev():
                    pltpu.make_async_copy(
                        bufs.at[b], out_slice(g - NBUF + b), osem.at[b]
                    ).wait()
                cps.append(pltpu.async_copy(
                    table_hbm.at[idx_v.at[g + b]], bufs.at[b], gsem.at[b]))
            for b in range(NBUF):
                cps[b].wait()
                pltpu.async_copy(bufs.at[b], out_slice(g + b), osem.at[b])
            return carry

        lax.fori_loop(0, n_chunks // NBUF, lambda i, c: ring_round(i * NBUF, c),
                      0, unroll=False)
        for b in range(NBUF):
            pltpu.make_async_copy(
                bufs.at[b], out_slice(n_chunks - NBUF + b), osem.at[b]).wait()

    return sc_gather


def _tc_ln_body(emb_ref, pe_ref, g_ref, b_ref, out_ref):
    e = emb_ref[...] + pe_ref[...]
    ones = jnp.full((D_MODEL, D_MODEL), 1.0 / D_MODEL, dtype=jnp.bfloat16)
    mean = lax.dot(e.astype(jnp.bfloat16), ones,
                   preferred_element_type=jnp.float32)
    c = e - mean
    var = lax.dot((c * c).astype(jnp.bfloat16), ones,
                  preferred_element_type=jnp.float32)
    out_ref[...] = c * lax.rsqrt(var + EPS) * g_ref[...] + b_ref[...]


def _tc_ln(emb, pe_big, gamma, beta):
    n_tok = emb.shape[0]
    rows = TC_SEQS * SEQ_LEN
    grid = n_tok // rows
    return pl.pallas_call(
        _tc_ln_body,
        grid=(grid,),
        in_specs=[
            pl.BlockSpec((rows, D_MODEL), lambda i: (i, 0)),
            pl.BlockSpec((rows, D_MODEL), lambda i: (0, 0)),
            pl.BlockSpec((1, D_MODEL), lambda i: (0, 0)),
            pl.BlockSpec((1, D_MODEL), lambda i: (0, 0)),
        ],
        out_specs=pl.BlockSpec((rows, D_MODEL), lambda i: (i, 0)),
        out_shape=jax.ShapeDtypeStruct((n_tok, D_MODEL), jnp.float32),
    )(emb, pe_big, gamma, beta)


def kernel(x, table, pe, gamma, beta):
    n_seq, seq_len = x.shape
    assert seq_len == SEQ_LEN
    n_tok = n_seq * seq_len
    info = plsc.get_sparse_core_info()
    nw = info.num_cores * info.num_subcores
    xc = x.astype(jnp.int32).reshape(nw, n_tok // nw // CHUNK, CHUNK)
    gathered = _make_sc_gather(n_tok)(xc, table)
    pe_big = jnp.tile(pe[0, :SEQ_LEN, :], (TC_SEQS, 1))
    out = _tc_ln(gathered, pe_big, gamma.reshape(1, D_MODEL),
                 beta.reshape(1, D_MODEL))
    return out.reshape(n_seq, seq_len, D_MODEL)
